# baseline (speedup 1.0000x reference)
import functools

import jax
import jax.numpy as jnp
from jax import lax
from jax.experimental import pallas as pl
from jax.experimental.pallas import tpu as pltpu
from jax.experimental.pallas import tpu_sc as plsc

_B = 16384
_NC = 2
_NS = 16
_NW = _NC * _NS
_BPW = _B // _NW
_CHUNK = 128
_NCH = _BPW // _CHUNK


def _sc_gather(idx2d, table_s2s2, table_trans):
    mesh = plsc.VectorSubcoreMesh(core_axis_name="c", subcore_axis_name="s")

    @functools.partial(
        pl.kernel,
        mesh=mesh,
        out_type=[
            jax.ShapeDtypeStruct((_B, 6), jnp.float32),
            jax.ShapeDtypeStruct((_B, 2), jnp.float32),
        ],
        scratch_types=[
            pltpu.VMEM((_NCH, _CHUNK), jnp.int32),
            pltpu.VMEM((_BPW, 6), jnp.float32),
            pltpu.VMEM((_BPW, 2), jnp.float32),
            pltpu.SemaphoreType.DMA,
        ],
        compiler_params=pltpu.CompilerParams(use_tc_tiling_on_sc=False),
    )
    def k(idx_hbm, s6_hbm, s2_hbm, out6_hbm, out2_hbm, idx_v, r6_v, r2_v, sem):
        wid = lax.axis_index("s") * _NC + lax.axis_index("c")
        base = wid * _BPW
        pltpu.sync_copy(idx_hbm.at[pl.ds(wid * _NCH, _NCH)], idx_v)
        copies = []
        for j in range(_NCH):
            copies.append(pltpu.async_copy(
                s6_hbm.at[idx_v.at[j]], r6_v.at[pl.ds(j * _CHUNK, _CHUNK)], sem))
            copies.append(pltpu.async_copy(
                s2_hbm.at[idx_v.at[j]], r2_v.at[pl.ds(j * _CHUNK, _CHUNK)], sem))
        for c in copies:
            c.wait()
        pltpu.sync_copy(r6_v, out6_hbm.at[pl.ds(base, _BPW)])
        pltpu.sync_copy(r2_v, out2_hbm.at[pl.ds(base, _BPW)])

    return k(idx2d, table_s2s2, table_trans)


def _rotmat_body(v_ref, out_ref):
    v = v_ref[...]
    v1 = v[:, 0:3]
    v2 = v[:, 3:6]
    n1 = jnp.sqrt(jnp.sum(v1 * v1, axis=-1, keepdims=True))
    e1 = v1 / n1
    dot = jnp.sum(e1 * v2, axis=-1, keepdims=True)
    u2 = v2 - dot * e1
    n2 = jnp.sqrt(jnp.sum(u2 * u2, axis=-1, keepdims=True))
    e2 = u2 / n2
    a0, a1, a2 = e1[:, 0:1], e1[:, 1:2], e1[:, 2:3]
    b0, b1, b2 = e2[:, 0:1], e2[:, 1:2], e2[:, 2:3]
    c0 = a1 * b2 - a2 * b1
    c1 = a2 * b0 - a0 * b2
    c2 = a0 * b1 - a1 * b0
    out_ref[...] = jnp.concatenate([e1, e2, c0, c1, c2], axis=-1)


def _tc_rotmat(rows6):
    blk = 2048
    out9 = pl.pallas_call(
        _rotmat_body,
        grid=(_B // blk,),
        in_specs=[pl.BlockSpec((blk, 6), lambda i: (i, 0))],
        out_specs=pl.BlockSpec((blk, 9), lambda i: (i, 0)),
        out_shape=jax.ShapeDtypeStruct((_B, 9), jnp.float32),
    )(rows6)
    return out9.reshape(_B, 3, 3)


def kernel(tilt_index, y, table_s2s2, table_trans):
    del y
    idx2d = tilt_index.reshape(_B // _CHUNK, _CHUNK)
    rows6, rows2 = _sc_gather(idx2d, table_s2s2, table_trans)
    R = _tc_rotmat(rows6)
    return (R, rows2)


# speedup vs baseline: 44.2279x; 44.2279x over previous
import functools

import jax
import jax.numpy as jnp
from jax import lax
from jax.experimental import pallas as pl
from jax.experimental.pallas import tpu as pltpu
from jax.experimental.pallas import tpu_sc as plsc

_B = 16384
_NC = 2
_NS = 16
_NW = _NC * _NS
_BPW = _B // _NW
_NG = _BPW // 16
_L = 16


def _rsqrt(x):
    i = plsc.bitcast(x, jnp.int32)
    i = jnp.int32(0x5F3759DF) - lax.shift_right_logical(i, 1)
    y = plsc.bitcast(i, jnp.float32)
    for _ in range(3):
        y = y * (1.5 - 0.5 * x * y * y)
    return y


def _sc_pose(tilt_index, t6, t2):
    mesh = plsc.VectorSubcoreMesh(core_axis_name="c", subcore_axis_name="s")

    @functools.partial(
        pl.kernel,
        mesh=mesh,
        out_type=[
            jax.ShapeDtypeStruct((9, _B), jnp.float32),
            jax.ShapeDtypeStruct((2, _B), jnp.float32),
        ],
        scratch_types=[
            pltpu.VMEM((_BPW,), jnp.int32),
            pltpu.VMEM((_BPW,), jnp.int32),
            pltpu.VMEM((6, _BPW * 8), jnp.float32),
            pltpu.VMEM((2, _BPW * 8), jnp.float32),
            pltpu.VMEM((9, _BPW), jnp.float32),
            pltpu.VMEM((2, _BPW), jnp.float32),
            pltpu.SemaphoreType.DMA,
            pltpu.SemaphoreType.DMA,
        ],
        compiler_params=pltpu.CompilerParams(needs_layout_passes=False),
    )
    def k(idx_hbm, t6_hbm, t2_hbm, o9_hbm, o2_hbm,
          idx_v, pos_v, blk6, blk2, b9, bt, sem_i, sem):
        wid = lax.axis_index("s") * _NC + lax.axis_index("c")
        base = wid * _BPW
        pltpu.async_copy(idx_hbm.at[pl.ds(base, _BPW)], idx_v, sem_i).wait()

        lane = jnp.arange(_L, dtype=jnp.int32)

        @pl.loop(0, _NG)
        def _prep(g):
            v = idx_v[pl.ds(g * _L, _L)]
            pos_v[pl.ds(g * _L, _L)] = (v & 7) + g * 128 + lane * 8

        @pl.loop(0, _NG)
        def _dma(ci):
            v = idx_v[pl.ds(ci * _L, _L)]
            handles = []
            for u in range(_L):
                a = pl.multiple_of(v[u] & jnp.int32(-8), 8)
                dst = pl.multiple_of((ci * _L + u) * 8, 8)
                handles.append(pltpu.async_copy(
                    t6_hbm.at[:, pl.ds(a, 8)], blk6.at[:, pl.ds(dst, 8)], sem))
                handles.append(pltpu.async_copy(
                    t2_hbm.at[:, pl.ds(a, 8)], blk2.at[:, pl.ds(dst, 8)], sem))
            for h in handles:
                h.wait()

        @pl.loop(0, _NG)
        def _math(g):
            p = pos_v[pl.ds(g * _L, _L)]

            def gat(tbl, j):
                return plsc.load_gather(
                    tbl, [jnp.full((_L,), j, jnp.int32), p])

            x0, x1, x2 = gat(blk6, 0), gat(blk6, 1), gat(blk6, 2)
            y0, y1, y2 = gat(blk6, 3), gat(blk6, 4), gat(blk6, 5)
            r1 = _rsqrt(x0 * x0 + x1 * x1 + x2 * x2)
            e10, e11, e12 = x0 * r1, x1 * r1, x2 * r1
            dot = e10 * y0 + e11 * y1 + e12 * y2
            u0, u1, u2 = y0 - dot * e10, y1 - dot * e11, y2 - dot * e12
            r2 = _rsqrt(u0 * u0 + u1 * u1 + u2 * u2)
            e20, e21, e22 = u0 * r2, u1 * r2, u2 * r2
            e30 = e11 * e22 - e12 * e21
            e31 = e12 * e20 - e10 * e22
            e32 = e10 * e21 - e11 * e20
            sl = pl.ds(g * _L, _L)
            b9[0, sl] = e10
            b9[1, sl] = e11
            b9[2, sl] = e12
            b9[3, sl] = e20
            b9[4, sl] = e21
            b9[5, sl] = e22
            b9[6, sl] = e30
            b9[7, sl] = e31
            b9[8, sl] = e32
            bt[0, sl] = gat(blk2, 0)
            bt[1, sl] = gat(blk2, 1)

        pltpu.sync_copy(b9, o9_hbm.at[:, pl.ds(base, _BPW)])
        pltpu.sync_copy(bt, o2_hbm.at[:, pl.ds(base, _BPW)])

    return k(tilt_index, t6, t2)


def kernel(tilt_index, y, table_s2s2, table_trans):
    del y
    out9, out2 = _sc_pose(tilt_index, table_s2s2.T, table_trans.T)
    return (out9.T.reshape(_B, 3, 3), out2.T)


# speedup vs baseline: 53.0857x; 1.2003x over previous
import functools

import jax
import jax.numpy as jnp
from jax import lax
from jax.experimental import pallas as pl
from jax.experimental.pallas import tpu as pltpu
from jax.experimental.pallas import tpu_sc as plsc

_B = 16384
_NC = 2
_NS = 16
_NW = _NC * _NS
_BPW = _B // _NW
_NG = _BPW // 16
_L = 16


def _rsqrt(x):
    i = plsc.bitcast(x, jnp.int32)
    i = jnp.int32(0x5F3759DF) - lax.shift_right_logical(i, 1)
    y = plsc.bitcast(i, jnp.float32)
    for _ in range(3):
        y = y * (1.5 - 0.5 * x * y * y)
    return y


def _sc_pose(tilt_index, t6, t2):
    mesh = plsc.VectorSubcoreMesh(core_axis_name="c", subcore_axis_name="s")

    @functools.partial(
        pl.kernel,
        mesh=mesh,
        out_type=[
            jax.ShapeDtypeStruct((9, _B), jnp.float32),
            jax.ShapeDtypeStruct((2, _B), jnp.float32),
        ],
        scratch_types=[
            pltpu.VMEM((_BPW,), jnp.int32),
            pltpu.VMEM((_BPW,), jnp.int32),
            pltpu.VMEM((6, _BPW * 8), jnp.float32),
            pltpu.VMEM((2, _BPW * 8), jnp.float32),
            pltpu.VMEM((9, _BPW), jnp.float32),
            pltpu.VMEM((2, _BPW), jnp.float32),
            pltpu.SemaphoreType.DMA,
            pltpu.SemaphoreType.DMA,
        ],
        compiler_params=pltpu.CompilerParams(needs_layout_passes=False),
    )
    def k(idx_hbm, t6_hbm, t2_hbm, o9_hbm, o2_hbm,
          idx_v, pos_v, blk6, blk2, b9, bt, sem_i, sem):
        wid = lax.axis_index("s") * _NC + lax.axis_index("c")
        base = wid * _BPW
        pltpu.async_copy(idx_hbm.at[pl.ds(base, _BPW)], idx_v, sem_i).wait()

        lane = jnp.arange(_L, dtype=jnp.int32)

        @pl.loop(0, _NG)
        def _prep(g):
            v = idx_v[pl.ds(g * _L, _L)]
            pos_v[pl.ds(g * _L, _L)] = (v & 7) + g * 128 + lane * 8

        def _issue(ci):
            v = idx_v[pl.ds(ci * _L, _L)]
            for u in range(_L):
                a = pl.multiple_of(v[u] & jnp.int32(-8), 8)
                dst = pl.multiple_of((ci * _L + u) * 8, 8)
                pltpu.async_copy(
                    t6_hbm.at[:, pl.ds(a, 8)], blk6.at[:, pl.ds(dst, 8)], sem)
                pltpu.async_copy(
                    t2_hbm.at[:, pl.ds(a, 8)], blk2.at[:, pl.ds(dst, 8)], sem)

        def _drain(ci):
            for u in range(_L):
                dst = pl.multiple_of((ci * _L + u) * 8, 8)
                pltpu.make_async_copy(
                    t6_hbm.at[:, pl.ds(0, 8)],
                    blk6.at[:, pl.ds(dst, 8)], sem).wait()
                pltpu.make_async_copy(
                    t2_hbm.at[:, pl.ds(0, 8)],
                    blk2.at[:, pl.ds(dst, 8)], sem).wait()

        def _math(g):
            p = pos_v[pl.ds(g * _L, _L)]

            def gat(tbl, j):
                return plsc.load_gather(
                    tbl, [jnp.full((_L,), j, jnp.int32), p])

            x0, x1, x2 = gat(blk6, 0), gat(blk6, 1), gat(blk6, 2)
            y0, y1, y2 = gat(blk6, 3), gat(blk6, 4), gat(blk6, 5)
            r1 = _rsqrt(x0 * x0 + x1 * x1 + x2 * x2)
            e10, e11, e12 = x0 * r1, x1 * r1, x2 * r1
            dot = e10 * y0 + e11 * y1 + e12 * y2
            u0, u1, u2 = y0 - dot * e10, y1 - dot * e11, y2 - dot * e12
            r2 = _rsqrt(u0 * u0 + u1 * u1 + u2 * u2)
            e20, e21, e22 = u0 * r2, u1 * r2, u2 * r2
            e30 = e11 * e22 - e12 * e21
            e31 = e12 * e20 - e10 * e22
            e32 = e10 * e21 - e11 * e20
            sl = pl.ds(g * _L, _L)
            b9[0, sl] = e10
            b9[1, sl] = e11
            b9[2, sl] = e12
            b9[3, sl] = e20
            b9[4, sl] = e21
            b9[5, sl] = e22
            b9[6, sl] = e30
            b9[7, sl] = e31
            b9[8, sl] = e32
            bt[0, sl] = gat(blk2, 0)
            bt[1, sl] = gat(blk2, 1)

        @pl.loop(0, _NG)
        def _dma(ci):
            _issue(ci)

            @pl.when(ci > 0)
            def _():
                _drain(ci - 1)
                _math(ci - 1)

        _drain(_NG - 1)
        _math(_NG - 1)

        pltpu.sync_copy(b9, o9_hbm.at[:, pl.ds(base, _BPW)])
        pltpu.sync_copy(bt, o2_hbm.at[:, pl.ds(base, _BPW)])

    return k(tilt_index, t6, t2)


def kernel(tilt_index, y, table_s2s2, table_trans):
    del y
    out9, out2 = _sc_pose(tilt_index, table_s2s2.T, table_trans.T)
    return (out9.T.reshape(_B, 3, 3), out2.T)


# speedup vs baseline: 55.9033x; 1.0531x over previous
import functools

import jax
import jax.numpy as jnp
from jax import lax
from jax.experimental import pallas as pl
from jax.experimental.pallas import tpu as pltpu
from jax.experimental.pallas import tpu_sc as plsc

_B = 16384
_NC = 2
_NS = 16
_NW = _NC * _NS
_BPW = _B // _NW
_NG = _BPW // 16
_L = 16


def _rsqrt(x):
    i = plsc.bitcast(x, jnp.int32)
    i = jnp.int32(0x5F3759DF) - lax.shift_right_logical(i, 1)
    y = plsc.bitcast(i, jnp.float32)
    for _ in range(3):
        y = y * (1.5 - 0.5 * x * y * y)
    return y


def _sc_pose(tilt_index, t6, t2):
    mesh = plsc.VectorSubcoreMesh(core_axis_name="c", subcore_axis_name="s")

    @functools.partial(
        pl.kernel,
        mesh=mesh,
        out_type=[
            jax.ShapeDtypeStruct((9, _B), jnp.float32),
            jax.ShapeDtypeStruct((2, _B), jnp.float32),
        ],
        scratch_types=[
            pltpu.VMEM((_BPW,), jnp.int32),
            pltpu.VMEM((_BPW,), jnp.int32),
            pltpu.VMEM((6, _BPW * 8), jnp.float32),
            pltpu.VMEM((2, _BPW * 8), jnp.float32),
            pltpu.VMEM((9, _BPW), jnp.float32),
            pltpu.VMEM((2, _BPW), jnp.float32),
            pltpu.SemaphoreType.DMA,
            pltpu.SemaphoreType.DMA,
        ],
        compiler_params=pltpu.CompilerParams(needs_layout_passes=False),
    )
    def k(idx_hbm, t6_hbm, t2_hbm, o9_hbm, o2_hbm,
          idx_v, pos_v, blk6, blk2, b9, bt, sem_i, sem):
        wid = lax.axis_index("s") * _NC + lax.axis_index("c")
        base = wid * _BPW
        pltpu.async_copy(idx_hbm.at[pl.ds(base, _BPW)], idx_v, sem_i).wait()

        lane = jnp.arange(_L, dtype=jnp.int32)

        @pl.loop(0, _NG)
        def _prep(g):
            v = idx_v[pl.ds(g * _L, _L)]
            pos_v[pl.ds(g * _L, _L)] = (v & 7) + g * 128 + lane * 8

        def _issue(ci):
            v = idx_v[pl.ds(ci * _L, _L)]
            for u in range(_L):
                a = pl.multiple_of(v[u] & jnp.int32(-8), 8)
                dst = pl.multiple_of((ci * _L + u) * 8, 8)
                pltpu.async_copy(
                    t6_hbm.at[:, pl.ds(a, 8)], blk6.at[:, pl.ds(dst, 8)], sem)
                pltpu.async_copy(
                    t2_hbm.at[:, pl.ds(a, 8)], blk2.at[:, pl.ds(dst, 8)], sem)

        def _drain(ci):
            for u in range(_L):
                dst = pl.multiple_of((ci * _L + u) * 8, 8)
                pltpu.make_async_copy(
                    t6_hbm.at[:, pl.ds(0, 8)],
                    blk6.at[:, pl.ds(dst, 8)], sem).wait()
                pltpu.make_async_copy(
                    t2_hbm.at[:, pl.ds(0, 8)],
                    blk2.at[:, pl.ds(dst, 8)], sem).wait()

        def _math(g):
            p = pos_v[pl.ds(g * _L, _L)]

            def gat(tbl, j):
                return plsc.load_gather(
                    tbl, [jnp.full((_L,), j, jnp.int32), p])

            x0, x1, x2 = gat(blk6, 0), gat(blk6, 1), gat(blk6, 2)
            y0, y1, y2 = gat(blk6, 3), gat(blk6, 4), gat(blk6, 5)
            r1 = _rsqrt(x0 * x0 + x1 * x1 + x2 * x2)
            e10, e11, e12 = x0 * r1, x1 * r1, x2 * r1
            dot = e10 * y0 + e11 * y1 + e12 * y2
            u0, u1, u2 = y0 - dot * e10, y1 - dot * e11, y2 - dot * e12
            r2 = _rsqrt(u0 * u0 + u1 * u1 + u2 * u2)
            e20, e21, e22 = u0 * r2, u1 * r2, u2 * r2
            e30 = e11 * e22 - e12 * e21
            e31 = e12 * e20 - e10 * e22
            e32 = e10 * e21 - e11 * e20
            sl = pl.ds(g * _L, _L)
            b9[0, sl] = e10
            b9[1, sl] = e11
            b9[2, sl] = e12
            b9[3, sl] = e20
            b9[4, sl] = e21
            b9[5, sl] = e22
            b9[6, sl] = e30
            b9[7, sl] = e31
            b9[8, sl] = e32
            bt[0, sl] = gat(blk2, 0)
            bt[1, sl] = gat(blk2, 1)

        _issue(0)

        @pl.loop(1, _NG)
        def _dma(ci):
            _issue(ci)

            @pl.when(ci > 1)
            def _():
                _drain(ci - 2)
                _math(ci - 2)

        _drain(_NG - 2)
        _math(_NG - 2)
        _drain(_NG - 1)
        _math(_NG - 1)

        pltpu.sync_copy(b9, o9_hbm.at[:, pl.ds(base, _BPW)])
        pltpu.sync_copy(bt, o2_hbm.at[:, pl.ds(base, _BPW)])

    return k(tilt_index, t6, t2)


def kernel(tilt_index, y, table_s2s2, table_trans):
    del y
    out9, out2 = _sc_pose(tilt_index, table_s2s2.T, table_trans.T)
    return (out9.T.reshape(_B, 3, 3), out2.T)
